# baseline (speedup 1.0000x reference)
import jax
import jax.numpy as jnp
from jax import lax
from jax.experimental import pallas as pl
from jax.experimental.pallas import tpu as pltpu
from jax.experimental.pallas import tpu_sc as plsc

_NC = 2
_NS = 16
_LANES = 16
_IDXW = 64
_SBROWS = 32
_NSLOT = 5
_LAG = 2


def _ceil_to(v, m):
    return (v + m - 1) // m * m


def _edge_agg(table, ixall, zblk, n_pad, num_ranges):
    d = table.shape[1]
    rsize = n_pad // num_ranges
    acc_rows = rsize + _NS
    ranges_per_core = num_ranges // _NC
    n_slabs = ixall.shape[0]
    slabs_per_tile = n_slabs // _NS
    share_z = acc_rows // _NS
    share_o = rsize // _NS

    def body(tab, ix, zb, out, acc, ixb, ldst, rows, *sems):
        gsem = sems[:_NSLOT]
        ssem = sems[_NSLOT:]
        c = lax.axis_index("c")
        s = lax.axis_index("s")
        for p in range(ranges_per_core):
            base = (c * ranges_per_core + p) * rsize
            pltpu.sync_copy(zb, rows.at[pl.ds(0, _IDXW * 2)])
            zoff = s * share_z
            for k in range(share_z // (_IDXW * 2)):
                pltpu.sync_copy(rows.at[pl.ds(0, _IDXW * 2)],
                                acc.at[pl.ds(zoff + k * _IDXW * 2, _IDXW * 2)])
            zrem = share_z % (_IDXW * 2)
            if zrem:
                pltpu.sync_copy(
                    rows.at[pl.ds(0, zrem)],
                    acc.at[pl.ds(zoff + (share_z // (_IDXW * 2)) * _IDXW * 2,
                                 zrem)])
            plsc.subcore_barrier()

            def slab_step(i, carry):
                pltpu.sync_copy(ix.at[s * slabs_per_tile + i], ixb)
                gd = [None] * _SBROWS
                sd = [None] * _SBROWS

                def fire_gather(r):
                    k = r % _NSLOT
                    gd[r] = pltpu.async_copy(
                        tab.at[ixb.at[r]],
                        rows.at[pl.ds(k * _IDXW, _IDXW)], gsem[k])
                    for v in range(_IDXW // _LANES):
                        dv = ixb[_SBROWS + r, pl.ds(v * _LANES, _LANES)]
                        loc = dv - base
                        okm = (loc >= 0) & (loc < rsize)
                        ldst[k, pl.ds(v * _LANES, _LANES)] = jnp.where(
                            okm, loc, rsize)

                def fire_scatter(r):
                    k = r % _NSLOT
                    gd[r].wait()
                    sd[r] = pltpu.async_copy(
                        rows.at[pl.ds(k * _IDXW, _IDXW)],
                        acc.at[ldst.at[k]], ssem[k], add=True)

                for r in range(_SBROWS):
                    if r >= _NSLOT:
                        sd[r - _NSLOT].wait()
                    fire_gather(r)
                    if r >= _LAG:
                        fire_scatter(r - _LAG)
                for r in range(_SBROWS - _LAG, _SBROWS):
                    fire_scatter(r)
                for r in range(_SBROWS - _NSLOT, _SBROWS):
                    sd[r].wait()
                return carry

            lax.fori_loop(0, slabs_per_tile, slab_step, 0)
            plsc.subcore_barrier()

            ooff = s * share_o
            stg = _IDXW * 2
            npiece = share_o // stg
            for k in range(npiece):
                pltpu.sync_copy(acc.at[pl.ds(ooff + k * stg, stg)],
                                rows.at[pl.ds(0, stg)])
                pltpu.sync_copy(rows.at[pl.ds(0, stg)],
                                out.at[pl.ds(base + ooff + k * stg, stg)])
            orem = share_o % stg
            if orem:
                pltpu.sync_copy(acc.at[pl.ds(ooff + npiece * stg, orem)],
                                rows.at[pl.ds(0, orem)])
                pltpu.sync_copy(
                    rows.at[pl.ds(0, orem)],
                    out.at[pl.ds(base + ooff + npiece * stg, orem)])
            plsc.subcore_barrier()

    f = pl.kernel(
        body,
        out_type=jax.ShapeDtypeStruct((n_pad, d), jnp.float32),
        mesh=plsc.VectorSubcoreMesh(core_axis_name="c", subcore_axis_name="s"),
        compiler_params=pltpu.CompilerParams(use_tc_tiling_on_sc=False),
        scratch_types=[
            pltpu.VMEM_SHARED((acc_rows, d), jnp.float32),
            pltpu.VMEM((2 * _SBROWS, _IDXW), jnp.int32),
            pltpu.VMEM((_NSLOT, _IDXW), jnp.int32),
            pltpu.VMEM((max(_NSLOT, 2) * _IDXW, d), jnp.float32),
        ] + [pltpu.SemaphoreType.DMA] * (2 * _NSLOT),
    )
    return f(table, ixall, zblk)


def _dense1(agg1, x, w1lT, b1, w1rT, blk):
    n, fin = x.shape
    h = w1lT.shape[1]
    grid = n // blk

    def body(a_ref, x_ref, wl_ref, b_ref, wr_ref, h_ref, d_ref):
        a = a_ref[...]
        sums = a[:, :fin]
        degc = jnp.maximum(a[:, fin:fin + 1], 1.0)
        mean = sums / degc
        acc = jax.lax.dot(mean, wl_ref[...],
                          preferred_element_type=jnp.float32)
        acc = acc + jax.lax.dot(x_ref[...], wr_ref[...],
                                preferred_element_type=jnp.float32)
        h_ref[...] = jnp.maximum(acc + b_ref[...], 0.0)
        d_ref[...] = degc

    return pl.pallas_call(
        body,
        grid=(grid,),
        in_specs=[
            pl.BlockSpec((blk, agg1.shape[1]), lambda i: (i, 0)),
            pl.BlockSpec((blk, fin), lambda i: (i, 0)),
            pl.BlockSpec((fin, h), lambda i: (0, 0)),
            pl.BlockSpec((1, h), lambda i: (0, 0)),
            pl.BlockSpec((fin, h), lambda i: (0, 0)),
        ],
        out_specs=[
            pl.BlockSpec((blk, h), lambda i: (i, 0)),
            pl.BlockSpec((blk, 1), lambda i: (i, 0)),
        ],
        out_shape=[
            jax.ShapeDtypeStruct((n, h), jnp.float32),
            jax.ShapeDtypeStruct((n, 1), jnp.float32),
        ],
    )(agg1, x, w1lT, b1, w1rT)


def _dense2(agg2, degc, h1, batch3d, w2lT, b2, w2rT, w3T, b3, blk, nb):
    n, h = h1.shape
    outd = w3T.shape[1]
    grid = n // blk

    def body(a_ref, d_ref, h1_ref, bt_ref, wl_ref, b2_ref, wr_ref, w3_ref,
             b3_ref, o_ref, pools, counts):
        i = pl.program_id(0)
        mean = a_ref[...] / d_ref[...]
        acc = jax.lax.dot(mean, wl_ref[...],
                          preferred_element_type=jnp.float32)
        acc = acc + jax.lax.dot(h1_ref[...], wr_ref[...],
                                preferred_element_type=jnp.float32)
        h2 = jnp.maximum(acc + b2_ref[...], 0.0)
        bb = bt_ref[...].reshape(1, blk)
        oh = (lax.broadcasted_iota(jnp.int32, (nb, blk), 0) == bb
              ).astype(jnp.float32)

        @pl.when(i == 0)
        def _():
            pools[...] = jnp.zeros_like(pools)
            counts[...] = jnp.zeros_like(counts)

        pools[...] += jax.lax.dot(oh, h2, preferred_element_type=jnp.float32)
        counts[...] += jnp.sum(oh, axis=1, keepdims=True)

        @pl.when(i == grid - 1)
        def _():
            pooled = pools[...] / jnp.maximum(counts[...], 1.0)
            o_ref[...] = jax.lax.dot(
                pooled, w3_ref[...],
                preferred_element_type=jnp.float32) + b3_ref[...]

    return pl.pallas_call(
        body,
        grid=(grid,),
        in_specs=[
            pl.BlockSpec((blk, h), lambda i: (i, 0)),
            pl.BlockSpec((blk, 1), lambda i: (i, 0)),
            pl.BlockSpec((blk, h), lambda i: (i, 0)),
            pl.BlockSpec((1, 1, blk), lambda i: (i, 0, 0)),
            pl.BlockSpec((h, h), lambda i: (0, 0)),
            pl.BlockSpec((1, h), lambda i: (0, 0)),
            pl.BlockSpec((h, h), lambda i: (0, 0)),
            pl.BlockSpec((h, outd), lambda i: (0, 0)),
            pl.BlockSpec((1, outd), lambda i: (0, 0)),
        ],
        out_specs=pl.BlockSpec((nb, outd), lambda i: (0, 0)),
        out_shape=jax.ShapeDtypeStruct((nb, outd), jnp.float32),
        scratch_shapes=[
            pltpu.VMEM((nb, h), jnp.float32),
            pltpu.VMEM((nb, 1), jnp.float32),
        ],
    )(agg2, degc, h1, batch3d, w2lT, b2, w2rT, w3T, b3)


def kernel(x, edge_index, batch, W1l, b1, W1r, W2l, b2, W2r, W3, b3):
    n, fin = x.shape
    e = edge_index.shape[1]
    h = W1l.shape[0]
    outd = W3.shape[0]
    nb = 128

    d1 = _ceil_to(fin + 1, 8)
    x_aug = jnp.concatenate(
        [x, jnp.ones((n, 1), jnp.float32),
         jnp.zeros((n, d1 - fin - 1), jnp.float32)], axis=1)

    sb_edges = _SBROWS * _IDXW
    epad = _ceil_to(e, _NS * sb_edges)
    src_p = jnp.concatenate(
        [edge_index[0], jnp.zeros((epad - e,), jnp.int32)]
    ).reshape(-1, _SBROWS, _IDXW)
    dst_p = jnp.concatenate(
        [edge_index[1], jnp.full((epad - e,), -1, jnp.int32)]
    ).reshape(-1, _SBROWS, _IDXW)
    ixall = jnp.concatenate([src_p, dst_p], axis=1)

    def n_ranges(d):
        r = _NC
        while (_ceil_to(-(-n // r), 128) + _NS) * d * 4 > 54 * 2**17:
            r += _NC
        return r

    r1 = n_ranges(d1)
    rs1 = _ceil_to(-(-n // r1), 128)
    np1 = rs1 * r1
    r2 = n_ranges(h)
    rs2 = _ceil_to(-(-n // r2), 128)
    np2 = rs2 * r2

    z1 = jnp.zeros((2 * _IDXW, d1), jnp.float32)
    z2 = jnp.zeros((2 * _IDXW, h), jnp.float32)

    agg1 = _edge_agg(x_aug, ixall, z1, np1, r1)[:n]

    blk = next(b for b in (1000, 500, 250, 200, 125, 100, 50, 25, 8, 1)
               if n % b == 0)
    h1, degc = _dense1(agg1, x, W1l.T, b1.reshape(1, -1), W1r.T, blk)

    agg2 = _edge_agg(h1, ixall, z2, np2, r2)[:n]

    batch3d = batch.reshape(n // blk, 1, blk)
    return _dense2(agg2, degc, h1, batch3d, W2l.T, b2.reshape(1, -1),
                   W2r.T, W3.T, b3.reshape(1, -1), blk, nb)


# speedup vs baseline: 1.0267x; 1.0267x over previous
import jax
import jax.numpy as jnp
from jax import lax
from jax.experimental import pallas as pl
from jax.experimental.pallas import tpu as pltpu
from jax.experimental.pallas import tpu_sc as plsc

_NC = 2
_NS = 16
_LANES = 16
_IDXW = 128


def _ceil_to(v, m):
    return (v + m - 1) // m * m


def _edge_agg(table, src2d, dst2d, zblk, n_pad, num_ranges, cpr):
    d = table.shape[1]
    rsize = n_pad // num_ranges
    acc_rows = _ceil_to(rsize + 1, _IDXW)
    ranges_per_core = num_ranges // _NC
    rows2d = src2d.shape[0]
    rows_per_tile = rows2d // _NS
    chunks = rows_per_tile // cpr
    chunk = cpr * _IDXW
    share_z = acc_rows // _NS
    share_o = rsize // _NS

    def body(tab, src, dst, zb, out, acc, idxb, dstb, ldst, rows, gsem, ssem):
        c = lax.axis_index("c")
        s = lax.axis_index("s")
        for p in range(ranges_per_core):
            base = (c * ranges_per_core + p) * rsize
            pltpu.sync_copy(zb, rows.at[pl.ds(0, _IDXW)])
            zoff = s * share_z
            for k in range(share_z // _IDXW):
                pltpu.sync_copy(rows.at[pl.ds(0, _IDXW)],
                                acc.at[pl.ds(zoff + k * _IDXW, _IDXW)])
            zrem = share_z % _IDXW
            if zrem:
                pltpu.sync_copy(
                    rows.at[pl.ds(0, zrem)],
                    acc.at[pl.ds(zoff + (share_z // _IDXW) * _IDXW, zrem)])
            plsc.subcore_barrier()

            def step(i, carry):
                row0 = s * rows_per_tile + i * cpr
                pltpu.sync_copy(src.at[pl.ds(row0, cpr)], idxb)
                pltpu.sync_copy(dst.at[pl.ds(row0, cpr)], dstb)
                gds = [pltpu.async_copy(tab.at[idxb.at[j]],
                                        rows.at[pl.ds(j * _IDXW, _IDXW)],
                                        gsem)
                       for j in range(cpr)]
                for j in range(cpr):
                    for k in range(_IDXW // _LANES):
                        v = dstb[j, pl.ds(k * _LANES, _LANES)]
                        loc = v - base
                        okm = (loc >= 0) & (loc < rsize)
                        ldst[j, pl.ds(k * _LANES, _LANES)] = jnp.where(
                            okm, loc, rsize)
                for g in gds:
                    g.wait()
                sds = [pltpu.async_copy(rows.at[pl.ds(j * _IDXW, _IDXW)],
                                        acc.at[ldst.at[j]], ssem, add=True)
                       for j in range(cpr)]
                for t in sds:
                    t.wait()
                return carry

            lax.fori_loop(0, chunks, step, 0)
            plsc.subcore_barrier()

            ooff = s * share_o
            npiece = share_o // _IDXW
            for k in range(npiece):
                pltpu.sync_copy(acc.at[pl.ds(ooff + k * _IDXW, _IDXW)],
                                rows.at[pl.ds(0, _IDXW)])
                pltpu.sync_copy(rows.at[pl.ds(0, _IDXW)],
                                out.at[pl.ds(base + ooff + k * _IDXW, _IDXW)])
            orem = share_o % _IDXW
            if orem:
                pltpu.sync_copy(acc.at[pl.ds(ooff + npiece * _IDXW, orem)],
                                rows.at[pl.ds(0, orem)])
                pltpu.sync_copy(
                    rows.at[pl.ds(0, orem)],
                    out.at[pl.ds(base + ooff + npiece * _IDXW, orem)])
            plsc.subcore_barrier()

    f = pl.kernel(
        body,
        out_type=jax.ShapeDtypeStruct((n_pad, d), jnp.float32),
        mesh=plsc.VectorSubcoreMesh(core_axis_name="c", subcore_axis_name="s"),
        compiler_params=pltpu.CompilerParams(use_tc_tiling_on_sc=False),
        scratch_types=[
            pltpu.VMEM_SHARED((acc_rows, d), jnp.float32),
            pltpu.VMEM((cpr, _IDXW), jnp.int32),
            pltpu.VMEM((cpr, _IDXW), jnp.int32),
            pltpu.VMEM((cpr, _IDXW), jnp.int32),
            pltpu.VMEM((chunk, d), jnp.float32),
            pltpu.SemaphoreType.DMA,
            pltpu.SemaphoreType.DMA,
        ],
    )
    return f(table, src2d, dst2d, zblk)


def _dense1(agg1, x, w1lT, b1, w1rT, blk):
    n, fin = x.shape
    h = w1lT.shape[1]
    grid = n // blk

    def body(a_ref, x_ref, wl_ref, b_ref, wr_ref, h_ref, d_ref):
        a = a_ref[...]
        sums = a[:, :fin]
        degc = jnp.maximum(a[:, fin:fin + 1], 1.0)
        mean = sums / degc
        acc = jax.lax.dot(mean, wl_ref[...],
                          preferred_element_type=jnp.float32)
        acc = acc + jax.lax.dot(x_ref[...], wr_ref[...],
                                preferred_element_type=jnp.float32)
        h_ref[...] = jnp.maximum(acc + b_ref[...], 0.0)
        d_ref[...] = degc

    return pl.pallas_call(
        body,
        grid=(grid,),
        in_specs=[
            pl.BlockSpec((blk, agg1.shape[1]), lambda i: (i, 0)),
            pl.BlockSpec((blk, fin), lambda i: (i, 0)),
            pl.BlockSpec((fin, h), lambda i: (0, 0)),
            pl.BlockSpec((1, h), lambda i: (0, 0)),
            pl.BlockSpec((fin, h), lambda i: (0, 0)),
        ],
        out_specs=[
            pl.BlockSpec((blk, h), lambda i: (i, 0)),
            pl.BlockSpec((blk, 1), lambda i: (i, 0)),
        ],
        out_shape=[
            jax.ShapeDtypeStruct((n, h), jnp.float32),
            jax.ShapeDtypeStruct((n, 1), jnp.float32),
        ],
    )(agg1, x, w1lT, b1, w1rT)


def _dense2(agg2, degc, h1, batch3d, w2lT, b2, w2rT, w3T, b3, blk, nb):
    n, h = h1.shape
    outd = w3T.shape[1]
    grid = n // blk

    def body(a_ref, d_ref, h1_ref, bt_ref, wl_ref, b2_ref, wr_ref, w3_ref,
             b3_ref, o_ref, pools, counts):
        i = pl.program_id(0)
        mean = a_ref[...] / d_ref[...]
        acc = jax.lax.dot(mean, wl_ref[...],
                          preferred_element_type=jnp.float32)
        acc = acc + jax.lax.dot(h1_ref[...], wr_ref[...],
                                preferred_element_type=jnp.float32)
        h2 = jnp.maximum(acc + b2_ref[...], 0.0)
        bb = bt_ref[...].reshape(1, blk)
        oh = (lax.broadcasted_iota(jnp.int32, (nb, blk), 0) == bb
              ).astype(jnp.float32)

        @pl.when(i == 0)
        def _():
            pools[...] = jnp.zeros_like(pools)
            counts[...] = jnp.zeros_like(counts)

        pools[...] += jax.lax.dot(oh, h2, preferred_element_type=jnp.float32)
        counts[...] += jnp.sum(oh, axis=1, keepdims=True)

        @pl.when(i == grid - 1)
        def _():
            pooled = pools[...] / jnp.maximum(counts[...], 1.0)
            o_ref[...] = jax.lax.dot(
                pooled, w3_ref[...],
                preferred_element_type=jnp.float32) + b3_ref[...]

    return pl.pallas_call(
        body,
        grid=(grid,),
        in_specs=[
            pl.BlockSpec((blk, h), lambda i: (i, 0)),
            pl.BlockSpec((blk, 1), lambda i: (i, 0)),
            pl.BlockSpec((blk, h), lambda i: (i, 0)),
            pl.BlockSpec((1, 1, blk), lambda i: (i, 0, 0)),
            pl.BlockSpec((h, h), lambda i: (0, 0)),
            pl.BlockSpec((1, h), lambda i: (0, 0)),
            pl.BlockSpec((h, h), lambda i: (0, 0)),
            pl.BlockSpec((h, outd), lambda i: (0, 0)),
            pl.BlockSpec((1, outd), lambda i: (0, 0)),
        ],
        out_specs=pl.BlockSpec((nb, outd), lambda i: (0, 0)),
        out_shape=jax.ShapeDtypeStruct((nb, outd), jnp.float32),
        scratch_shapes=[
            pltpu.VMEM((nb, h), jnp.float32),
            pltpu.VMEM((nb, 1), jnp.float32),
        ],
    )(agg2, degc, h1, batch3d, w2lT, b2, w2rT, w3T, b3)


def kernel(x, edge_index, batch, W1l, b1, W1r, W2l, b2, W2r, W3, b3):
    n, fin = x.shape
    e = edge_index.shape[1]
    h = W1l.shape[0]
    outd = W3.shape[0]
    nb = 128

    d1 = _ceil_to(fin + 1, 8)
    x_aug = jnp.concatenate(
        [x, jnp.ones((n, 1), jnp.float32),
         jnp.zeros((n, d1 - fin - 1), jnp.float32)], axis=1)

    epad = _ceil_to(e, _NS * _IDXW * 16)
    src_p = jnp.concatenate(
        [edge_index[0], jnp.zeros((epad - e,), jnp.int32)]).reshape(-1, _IDXW)
    dst_p = jnp.concatenate(
        [edge_index[1], jnp.full((epad - e,), -1, jnp.int32)]
    ).reshape(-1, _IDXW)

    def n_ranges(d):
        r = _NC
        while (_ceil_to(_ceil_to(-(-n // r), _IDXW) + 1, _IDXW)) * d * 4 \
                > 7 * 2**20:
            r += _NC
        return r

    r1 = n_ranges(d1)
    rs1 = _ceil_to(-(-n // r1), _IDXW)
    np1 = rs1 * r1
    r2 = n_ranges(h)
    rs2 = _ceil_to(-(-n // r2), _IDXW)
    np2 = rs2 * r2

    z1 = jnp.zeros((_IDXW, d1), jnp.float32)
    z2 = jnp.zeros((_IDXW, h), jnp.float32)

    agg1 = _edge_agg(x_aug, src_p, dst_p, z1, np1, r1, 8)[:n]

    blk = next(b for b in (1000, 500, 250, 200, 125, 100, 50, 25, 8, 1)
               if n % b == 0)
    h1, degc = _dense1(agg1, x, W1l.T, b1.reshape(1, -1), W1r.T, blk)

    agg2 = _edge_agg(h1, src_p, dst_p, z2, np2, r2, 2)[:n]

    batch3d = batch.reshape(n // blk, 1, blk)
    return _dense2(agg2, degc, h1, batch3d, W2l.T, b2.reshape(1, -1),
                   W2r.T, W3.T, b3.reshape(1, -1), blk, nb)


# speedup vs baseline: 1.3421x; 1.3072x over previous
import jax
import jax.numpy as jnp
from jax import lax
from jax.experimental import pallas as pl
from jax.experimental.pallas import tpu as pltpu
from jax.experimental.pallas import tpu_sc as plsc

_NC = 2
_NS = 16
_LANES = 16
_IDXW = 128


def _ceil_to(v, m):
    return (v + m - 1) // m * m


def _edge_agg(table, src2d, dst2d, zblk, n_pad, num_ranges, cpr):
    d = table.shape[1]
    rsize = n_pad // num_ranges
    acc_rows = _ceil_to(rsize + 1, _IDXW)
    ranges_per_core = num_ranges // _NC
    rows2d = src2d.shape[0]
    rows_per_tile = rows2d // _NS
    chunks = rows_per_tile // cpr
    chunk = cpr * _IDXW
    share_z = acc_rows // _NS
    share_o = rsize // _NS

    def body(tab, src, dst, zb, out, acc, idxb, dstb, ldst, rows, gsem, ssem):
        c = lax.axis_index("c")
        s = lax.axis_index("s")
        for p in range(ranges_per_core):
            base = (c * ranges_per_core + p) * rsize
            pltpu.sync_copy(zb, rows.at[pl.ds(0, _IDXW)])
            zoff = s * share_z
            for k in range(share_z // _IDXW):
                pltpu.sync_copy(rows.at[pl.ds(0, _IDXW)],
                                acc.at[pl.ds(zoff + k * _IDXW, _IDXW)])
            zrem = share_z % _IDXW
            if zrem:
                pltpu.sync_copy(
                    rows.at[pl.ds(0, zrem)],
                    acc.at[pl.ds(zoff + (share_z // _IDXW) * _IDXW, zrem)])
            plsc.subcore_barrier()

            def step(i, carry):
                row0 = s * rows_per_tile + i * cpr
                pltpu.sync_copy(src.at[pl.ds(row0, cpr)], idxb)
                pltpu.sync_copy(dst.at[pl.ds(row0, cpr)], dstb)
                gds = [pltpu.async_copy(tab.at[idxb.at[j]],
                                        rows.at[pl.ds(j * _IDXW, _IDXW)],
                                        gsem)
                       for j in range(cpr)]
                for j in range(cpr):
                    for k in range(_IDXW // _LANES):
                        v = dstb[j, pl.ds(k * _LANES, _LANES)]
                        loc = v - base
                        okm = (loc >= 0) & (loc < rsize)
                        ldst[j, pl.ds(k * _LANES, _LANES)] = jnp.where(
                            okm, loc, rsize)
                for g in gds:
                    g.wait()
                sds = [pltpu.async_copy(rows.at[pl.ds(j * _IDXW, _IDXW)],
                                        acc.at[ldst.at[j]], ssem, add=True)
                       for j in range(cpr)]
                for t in sds:
                    t.wait()
                return carry

            lax.fori_loop(0, chunks, step, 0)
            plsc.subcore_barrier()

            ooff = s * share_o
            npiece = share_o // _IDXW
            for k in range(npiece):
                pltpu.sync_copy(acc.at[pl.ds(ooff + k * _IDXW, _IDXW)],
                                rows.at[pl.ds(0, _IDXW)])
                pltpu.sync_copy(rows.at[pl.ds(0, _IDXW)],
                                out.at[pl.ds(base + ooff + k * _IDXW, _IDXW)])
            orem = share_o % _IDXW
            if orem:
                pltpu.sync_copy(acc.at[pl.ds(ooff + npiece * _IDXW, orem)],
                                rows.at[pl.ds(0, orem)])
                pltpu.sync_copy(
                    rows.at[pl.ds(0, orem)],
                    out.at[pl.ds(base + ooff + npiece * _IDXW, orem)])
            plsc.subcore_barrier()

    f = pl.kernel(
        body,
        out_type=jax.ShapeDtypeStruct((n_pad, d), jnp.float32),
        mesh=plsc.VectorSubcoreMesh(core_axis_name="c", subcore_axis_name="s"),
        compiler_params=pltpu.CompilerParams(use_tc_tiling_on_sc=False),
        scratch_types=[
            pltpu.VMEM_SHARED((acc_rows, d), jnp.float32),
            pltpu.VMEM((cpr, _IDXW), jnp.int32),
            pltpu.VMEM((cpr, _IDXW), jnp.int32),
            pltpu.VMEM((cpr, _IDXW), jnp.int32),
            pltpu.VMEM((chunk, d), jnp.float32),
            pltpu.SemaphoreType.DMA,
            pltpu.SemaphoreType.DMA,
        ],
    )
    return f(table, src2d, dst2d, zblk)


def _dense1(agg1, x, w1lT, b1, w1rT, blk):
    n, fin = x.shape
    h = w1lT.shape[1]
    grid = n // blk

    def body(a_ref, x_ref, wl_ref, b_ref, wr_ref, h_ref, d_ref):
        a = a_ref[...]
        sums = a[:, :fin]
        degc = jnp.maximum(a[:, fin:fin + 1], 1.0)
        mean = sums / degc
        acc = jax.lax.dot(mean, wl_ref[...],
                          preferred_element_type=jnp.float32)
        acc = acc + jax.lax.dot(x_ref[...], wr_ref[...],
                                preferred_element_type=jnp.float32)
        h_ref[...] = jnp.maximum(acc + b_ref[...], 0.0)
        d_ref[...] = degc

    return pl.pallas_call(
        body,
        grid=(grid,),
        in_specs=[
            pl.BlockSpec((blk, agg1.shape[1]), lambda i: (i, 0)),
            pl.BlockSpec((blk, fin), lambda i: (i, 0)),
            pl.BlockSpec((fin, h), lambda i: (0, 0)),
            pl.BlockSpec((1, h), lambda i: (0, 0)),
            pl.BlockSpec((fin, h), lambda i: (0, 0)),
        ],
        out_specs=[
            pl.BlockSpec((blk, h), lambda i: (i, 0)),
            pl.BlockSpec((blk, 1), lambda i: (i, 0)),
        ],
        out_shape=[
            jax.ShapeDtypeStruct((n, h), jnp.float32),
            jax.ShapeDtypeStruct((n, 1), jnp.float32),
        ],
    )(agg1, x, w1lT, b1, w1rT)


def _dense2(agg2, degc, h1, batch3d, w2lT, b2, w2rT, w3T, b3, blk, nb):
    n, h = h1.shape
    outd = w3T.shape[1]
    grid = n // blk

    def body(a_ref, d_ref, h1_ref, bt_ref, wl_ref, b2_ref, wr_ref, w3_ref,
             b3_ref, o_ref, pools, counts):
        i = pl.program_id(0)
        mean = a_ref[...] / d_ref[...]
        acc = jax.lax.dot(mean, wl_ref[...],
                          preferred_element_type=jnp.float32)
        acc = acc + jax.lax.dot(h1_ref[...], wr_ref[...],
                                preferred_element_type=jnp.float32)
        h2 = jnp.maximum(acc + b2_ref[...], 0.0)
        bb = bt_ref[...].reshape(1, blk)
        oh = (lax.broadcasted_iota(jnp.int32, (nb, blk), 0) == bb
              ).astype(jnp.float32)

        @pl.when(i == 0)
        def _():
            pools[...] = jnp.zeros_like(pools)
            counts[...] = jnp.zeros_like(counts)

        pools[...] += jax.lax.dot(oh, h2, preferred_element_type=jnp.float32)
        counts[...] += jnp.sum(oh, axis=1, keepdims=True)

        @pl.when(i == grid - 1)
        def _():
            pooled = pools[...] / jnp.maximum(counts[...], 1.0)
            o_ref[...] = jax.lax.dot(
                pooled, w3_ref[...],
                preferred_element_type=jnp.float32) + b3_ref[...]

    return pl.pallas_call(
        body,
        grid=(grid,),
        in_specs=[
            pl.BlockSpec((blk, h), lambda i: (i, 0)),
            pl.BlockSpec((blk, 1), lambda i: (i, 0)),
            pl.BlockSpec((blk, h), lambda i: (i, 0)),
            pl.BlockSpec((1, 1, blk), lambda i: (i, 0, 0)),
            pl.BlockSpec((h, h), lambda i: (0, 0)),
            pl.BlockSpec((1, h), lambda i: (0, 0)),
            pl.BlockSpec((h, h), lambda i: (0, 0)),
            pl.BlockSpec((h, outd), lambda i: (0, 0)),
            pl.BlockSpec((1, outd), lambda i: (0, 0)),
        ],
        out_specs=pl.BlockSpec((nb, outd), lambda i: (0, 0)),
        out_shape=jax.ShapeDtypeStruct((nb, outd), jnp.float32),
        scratch_shapes=[
            pltpu.VMEM((nb, h), jnp.float32),
            pltpu.VMEM((nb, 1), jnp.float32),
        ],
    )(agg2, degc, h1, batch3d, w2lT, b2, w2rT, w3T, b3)


def kernel(x, edge_index, batch, W1l, b1, W1r, W2l, b2, W2r, W3, b3):
    n, fin = x.shape
    e = edge_index.shape[1]
    h = W1l.shape[0]
    outd = W3.shape[0]
    nb = 128

    d1 = _ceil_to(fin + 1, 8)
    x_aug = jnp.concatenate(
        [x, jnp.ones((n, 1), jnp.float32),
         jnp.zeros((n, d1 - fin - 1), jnp.float32)], axis=1)

    epad = _ceil_to(e, _NS * _IDXW * 16)
    src_p = jnp.concatenate(
        [edge_index[0], jnp.zeros((epad - e,), jnp.int32)]).reshape(-1, _IDXW)
    dst_p = jnp.concatenate(
        [edge_index[1], jnp.full((epad - e,), -1, jnp.int32)]
    ).reshape(-1, _IDXW)

    def n_ranges(d):
        r = _NC
        while (_ceil_to(_ceil_to(-(-n // r), _IDXW) + 1, _IDXW)) * d * 4 \
                > 7 * 2**20:
            r += _NC
        return r

    r1 = n_ranges(d1)
    rs1 = _ceil_to(-(-n // r1), _IDXW)
    np1 = rs1 * r1
    h2w = h // 2
    r2 = n_ranges(h2w)
    rs2 = _ceil_to(-(-n // r2), _IDXW)
    np2 = rs2 * r2

    z1 = jnp.zeros((_IDXW, d1), jnp.float32)
    z2 = jnp.zeros((_IDXW, h2w), jnp.float32)

    agg1 = _edge_agg(x_aug, src_p, dst_p, z1, np1, r1, 8)[:n]

    blk = next(b for b in (1000, 500, 250, 200, 125, 100, 50, 25, 8, 1)
               if n % b == 0)
    h1, degc = _dense1(agg1, x, W1l.T, b1.reshape(1, -1), W1r.T, blk)

    agg2a = _edge_agg(h1[:, :h2w], src_p, dst_p, z2, np2, r2, 4)[:n]
    agg2b = _edge_agg(h1[:, h2w:], src_p, dst_p, z2, np2, r2, 4)[:n]
    agg2 = jnp.concatenate([agg2a, agg2b], axis=1)

    batch3d = batch.reshape(n // blk, 1, blk)
    return _dense2(agg2, degc, h1, batch3d, W2l.T, b2.reshape(1, -1),
                   W2r.T, W3.T, b3.reshape(1, -1), blk, nb)


# speedup vs baseline: 1.8741x; 1.3964x over previous
import jax
import jax.numpy as jnp
from jax import lax
from jax.experimental import pallas as pl
from jax.experimental.pallas import tpu as pltpu
from jax.experimental.pallas import tpu_sc as plsc

_NC = 2
_NS = 16
_LANES = 16
_IDXW = 128


def _ceil_to(v, m):
    return (v + m - 1) // m * m


def _edge_agg(table, src2d, dst2d, zblk, n_pad):
    d = table.shape[1]
    acc_rows = _ceil_to(n_pad + 1, _IDXW)
    rows2d = src2d.shape[0]
    rows_per_tile = rows2d // (_NC * _NS)
    cpr = 8 if d <= 12 else 4
    chunks = rows_per_tile // cpr
    chunk = cpr * _IDXW
    share_z = acc_rows // _NS
    share_o = n_pad // _NS

    def body(tab, src, dst, zb, out, acc, idxb, dstb, ldst, rows, gsem, ssem):
        c = lax.axis_index("c")
        s = lax.axis_index("s")
        pltpu.sync_copy(zb, rows.at[pl.ds(0, _IDXW)])
        zoff = s * share_z
        for k in range(share_z // _IDXW):
            pltpu.sync_copy(rows.at[pl.ds(0, _IDXW)],
                            acc.at[pl.ds(zoff + k * _IDXW, _IDXW)])
        zrem = share_z % _IDXW
        if zrem:
            pltpu.sync_copy(
                rows.at[pl.ds(0, zrem)],
                acc.at[pl.ds(zoff + (share_z // _IDXW) * _IDXW, zrem)])
        plsc.subcore_barrier()

        def step(i, carry):
            row0 = (c * _NS + s) * rows_per_tile + i * cpr
            pltpu.sync_copy(src.at[pl.ds(row0, cpr)], idxb)
            pltpu.sync_copy(dst.at[pl.ds(row0, cpr)], dstb)
            gds = [pltpu.async_copy(tab.at[idxb.at[j]],
                                    rows.at[pl.ds(j * _IDXW, _IDXW)],
                                    gsem)
                   for j in range(cpr)]
            for j in range(cpr):
                for k in range(_IDXW // _LANES):
                    v = dstb[j, pl.ds(k * _LANES, _LANES)]
                    okm = (v >= 0) & (v < n_pad)
                    ldst[j, pl.ds(k * _LANES, _LANES)] = jnp.where(
                        okm, v, n_pad)
            for g in gds:
                g.wait()
            sds = [pltpu.async_copy(rows.at[pl.ds(j * _IDXW, _IDXW)],
                                    acc.at[ldst.at[j]], ssem, add=True)
                   for j in range(cpr)]
            for t in sds:
                t.wait()
            return carry

        lax.fori_loop(0, chunks, step, 0)
        plsc.subcore_barrier()

        obase = c * n_pad
        ooff = s * share_o
        npiece = share_o // _IDXW
        for k in range(npiece):
            pltpu.sync_copy(acc.at[pl.ds(ooff + k * _IDXW, _IDXW)],
                            rows.at[pl.ds(0, _IDXW)])
            pltpu.sync_copy(rows.at[pl.ds(0, _IDXW)],
                            out.at[pl.ds(obase + ooff + k * _IDXW, _IDXW)])
        orem = share_o % _IDXW
        if orem:
            pltpu.sync_copy(acc.at[pl.ds(ooff + npiece * _IDXW, orem)],
                            rows.at[pl.ds(0, orem)])
            pltpu.sync_copy(
                rows.at[pl.ds(0, orem)],
                out.at[pl.ds(obase + ooff + npiece * _IDXW, orem)])
        plsc.subcore_barrier()

    f = pl.kernel(
        body,
        out_type=jax.ShapeDtypeStruct((_NC * n_pad, d), jnp.float32),
        mesh=plsc.VectorSubcoreMesh(core_axis_name="c", subcore_axis_name="s"),
        compiler_params=pltpu.CompilerParams(use_tc_tiling_on_sc=False),
        scratch_types=[
            pltpu.VMEM_SHARED((acc_rows, d), jnp.float32),
            pltpu.VMEM((cpr, _IDXW), jnp.int32),
            pltpu.VMEM((cpr, _IDXW), jnp.int32),
            pltpu.VMEM((cpr, _IDXW), jnp.int32),
            pltpu.VMEM((chunk, d), jnp.float32),
            pltpu.SemaphoreType.DMA,
            pltpu.SemaphoreType.DMA,
        ],
    )
    o = f(table, src2d, dst2d, zblk)
    return o[:n_pad] + o[n_pad:]


def _dense1(agg1, x, w1lT, b1, w1rT, blk):
    n, fin = x.shape
    h = w1lT.shape[1]
    grid = n // blk

    def body(a_ref, x_ref, wl_ref, b_ref, wr_ref, h_ref, d_ref):
        a = a_ref[...]
        sums = a[:, :fin]
        degc = jnp.maximum(a[:, fin:fin + 1], 1.0)
        mean = sums / degc
        acc = jax.lax.dot(mean, wl_ref[...],
                          preferred_element_type=jnp.float32)
        acc = acc + jax.lax.dot(x_ref[...], wr_ref[...],
                                preferred_element_type=jnp.float32)
        h_ref[...] = jnp.maximum(acc + b_ref[...], 0.0)
        d_ref[...] = degc

    return pl.pallas_call(
        body,
        grid=(grid,),
        in_specs=[
            pl.BlockSpec((blk, agg1.shape[1]), lambda i: (i, 0)),
            pl.BlockSpec((blk, fin), lambda i: (i, 0)),
            pl.BlockSpec((fin, h), lambda i: (0, 0)),
            pl.BlockSpec((1, h), lambda i: (0, 0)),
            pl.BlockSpec((fin, h), lambda i: (0, 0)),
        ],
        out_specs=[
            pl.BlockSpec((blk, h), lambda i: (i, 0)),
            pl.BlockSpec((blk, 1), lambda i: (i, 0)),
        ],
        out_shape=[
            jax.ShapeDtypeStruct((n, h), jnp.float32),
            jax.ShapeDtypeStruct((n, 1), jnp.float32),
        ],
    )(agg1, x, w1lT, b1, w1rT)


def _dense2(agg2, degc, h1, batch3d, w2lT, b2, w2rT, w3T, b3, blk, nb):
    n, h = h1.shape
    outd = w3T.shape[1]
    grid = n // blk

    def body(a_ref, d_ref, h1_ref, bt_ref, wl_ref, b2_ref, wr_ref, w3_ref,
             b3_ref, o_ref, pools, counts):
        i = pl.program_id(0)
        mean = a_ref[...] / d_ref[...]
        acc = jax.lax.dot(mean, wl_ref[...],
                          preferred_element_type=jnp.float32)
        acc = acc + jax.lax.dot(h1_ref[...], wr_ref[...],
                                preferred_element_type=jnp.float32)
        h2 = jnp.maximum(acc + b2_ref[...], 0.0)
        bb = bt_ref[...].reshape(1, blk)
        oh = (lax.broadcasted_iota(jnp.int32, (nb, blk), 0) == bb
              ).astype(jnp.float32)

        @pl.when(i == 0)
        def _():
            pools[...] = jnp.zeros_like(pools)
            counts[...] = jnp.zeros_like(counts)

        pools[...] += jax.lax.dot(oh, h2, preferred_element_type=jnp.float32)
        counts[...] += jnp.sum(oh, axis=1, keepdims=True)

        @pl.when(i == grid - 1)
        def _():
            pooled = pools[...] / jnp.maximum(counts[...], 1.0)
            o_ref[...] = jax.lax.dot(
                pooled, w3_ref[...],
                preferred_element_type=jnp.float32) + b3_ref[...]

    return pl.pallas_call(
        body,
        grid=(grid,),
        in_specs=[
            pl.BlockSpec((blk, h), lambda i: (i, 0)),
            pl.BlockSpec((blk, 1), lambda i: (i, 0)),
            pl.BlockSpec((blk, h), lambda i: (i, 0)),
            pl.BlockSpec((1, 1, blk), lambda i: (i, 0, 0)),
            pl.BlockSpec((h, h), lambda i: (0, 0)),
            pl.BlockSpec((1, h), lambda i: (0, 0)),
            pl.BlockSpec((h, h), lambda i: (0, 0)),
            pl.BlockSpec((h, outd), lambda i: (0, 0)),
            pl.BlockSpec((1, outd), lambda i: (0, 0)),
        ],
        out_specs=pl.BlockSpec((nb, outd), lambda i: (0, 0)),
        out_shape=jax.ShapeDtypeStruct((nb, outd), jnp.float32),
        scratch_shapes=[
            pltpu.VMEM((nb, h), jnp.float32),
            pltpu.VMEM((nb, 1), jnp.float32),
        ],
    )(agg2, degc, h1, batch3d, w2lT, b2, w2rT, w3T, b3)


def kernel(x, edge_index, batch, W1l, b1, W1r, W2l, b2, W2r, W3, b3):
    n, fin = x.shape
    e = edge_index.shape[1]
    h = W1l.shape[0]
    outd = W3.shape[0]
    nb = 128

    w1 = 12
    np_ = _ceil_to(n, _IDXW)
    t1a = jnp.concatenate(
        [x[:, :w1], jnp.zeros((np_ - n, w1), jnp.float32)], axis=0)
    t1b = jnp.concatenate([
        jnp.concatenate([x[:, w1:], jnp.ones((n, 1), jnp.float32),
                         jnp.zeros((n, 2 * w1 - fin - 1), jnp.float32)],
                        axis=1),
        jnp.zeros((np_ - n, w1), jnp.float32)], axis=0)

    epad = _ceil_to(e, _NC * _NS * _IDXW * 8)
    src_p = jnp.concatenate(
        [edge_index[0], jnp.zeros((epad - e,), jnp.int32)]).reshape(-1, _IDXW)
    dst_p = jnp.concatenate(
        [edge_index[1], jnp.full((epad - e,), -1, jnp.int32)]
    ).reshape(-1, _IDXW)

    z12 = jnp.zeros((_IDXW, w1), jnp.float32)
    a1a = _edge_agg(t1a, src_p, dst_p, z12, np_)[:n]
    a1b = _edge_agg(t1b, src_p, dst_p, z12, np_)[:n]
    agg1 = jnp.concatenate([a1a, a1b[:, :fin - w1 + 1]], axis=1)

    blk = next(b for b in (1000, 500, 250, 200, 125, 100, 50, 25, 8, 1)
               if n % b == 0)
    h1, degc = _dense1(agg1, x, W1l.T, b1.reshape(1, -1), W1r.T, blk)

    w2 = 16
    z16 = jnp.zeros((_IDXW, w2), jnp.float32)
    h1p = jnp.concatenate(
        [h1, jnp.zeros((np_ - n, h), jnp.float32)], axis=0)
    parts = [_edge_agg(h1p[:, q * w2:(q + 1) * w2], src_p, dst_p, z16,
                       np_)[:n]
             for q in range(h // w2)]
    agg2 = jnp.concatenate(parts, axis=1)

    batch3d = batch.reshape(n // blk, 1, blk)
    return _dense2(agg2, degc, h1, batch3d, W2l.T, b2.reshape(1, -1),
                   W2r.T, W3.T, b3.reshape(1, -1), blk, nb)
